# baseline (speedup 1.0000x reference)
import functools

import jax
import jax.numpy as jnp
from jax import lax
from jax.experimental import pallas as pl
from jax.experimental.pallas import tpu as pltpu
from jax.experimental.pallas import tpu_sc as plsc

N_NODES = 10000
N_TBL = 10016
N_EDGES = 320000
D = 128

CG = 16
NG = D // CG
CHUNK = 128
ROWS = N_EDGES // CHUNK
ROWS_PT = 158
ROWS_PAD = 16 * ROWS_PT
E_PAD = ROWS_PAD * CHUNK
N_PAD = 10240
STRIPE = N_PAD // 16

BE = 512
NEB = E_PAD // BE


def _node_mm_kernel(h_ref, w_ref, b_ref, a1_ref, a2_ref, a3_ref, b1_ref,
                    b2_ref):
    h = h_ref[...]
    outs = (a1_ref, a2_ref, a3_ref, b1_ref, b2_ref)
    for i, o_ref in enumerate(outs):
        w = w_ref[i]
        o_ref[...] = jax.lax.dot_general(
            h, w, (((1,), (1,)), ((), ())),
            preferred_element_type=jnp.float32) + b_ref[i]


def _edge_pack_kernel(e_ref, w_ref, b_ref, cp_ref, ep_ref):
    i = pl.program_id(0)

    @pl.when(i < N_EDGES // BE)
    def _():
        ev = e_ref[...]
        c = jax.lax.dot_general(ev, w_ref[...], (((1,), (1,)), ((), ())),
                                preferred_element_type=jnp.float32) + b_ref[...]
        cp_ref[...] = c.reshape(BE, NG, CG).transpose(1, 0, 2)
        ep_ref[...] = ev.reshape(BE, NG, CG).transpose(1, 0, 2)

    @pl.when(i >= N_EDGES // BE)
    def _():
        cp_ref[...] = jnp.zeros((NG, BE, CG), jnp.float32)
        ep_ref[...] = jnp.zeros((NG, BE, CG), jnp.float32)


def _final_node_kernel(a1_ref, nf_ref, df_ref, nb_ref, db_ref, h_ref,
                       gamma_ref, beta_ref, out_ref):
    eps = 1e-6
    hf = nf_ref[...] / (df_ref[...] + eps)
    hb = nb_ref[...] / (db_ref[...] + eps)
    x = a1_ref[...] + hf + hb
    mu = jnp.mean(x, axis=0, keepdims=True)
    var = jnp.mean((x - mu) ** 2, axis=0, keepdims=True)
    xn = gamma_ref[...] * (x - mu) / jnp.sqrt(var + 1e-5) + beta_ref[...]
    out_ref[...] = jnp.maximum(xn, 0.0) + h_ref[...]


def _stats_body(cp_hbm, b1p_hbm, b2p_hbm, src2_hbm, dst2_hbm,
                xp_hbm, st_hbm,
                ixs0, ixd0, b1b0, b2b0, cb0, xb0,
                ixs1, ixd1, b1b1, b2b1, cb1, xb1,
                stb, sem_i0, sem_i1, sem_g0, sem_g1, sem_w0, sem_w1):
    c = lax.axis_index("c")
    s = lax.axis_index("s")
    base = s * ROWS_PT
    sets = ((ixs0, ixd0, b1b0, b2b0, cb0, xb0, sem_i0, sem_g0, sem_w0),
            (ixs1, ixd1, b1b1, b2b1, cb1, xb1, sem_i1, sem_g1, sem_w1))

    def issue_idx(k, st_):
        ixs, ixd = st_[0], st_[1]
        pltpu.async_copy(src2_hbm.at[base + k], ixs, st_[6])
        pltpu.async_copy(dst2_hbm.at[base + k], ixd, st_[6])

    def bias_and_gather(k, gg, st_):
        ixs, ixd, b1b, b2b = st_[0], st_[1], st_[2], st_[3]
        pltpu.make_async_copy(src2_hbm.at[base + k], ixs, st_[6]).wait()
        pltpu.make_async_copy(dst2_hbm.at[base + k], ixd, st_[6]).wait()
        bias = gg * N_TBL
        for i in range(CHUNK // 16):
            sl = pl.ds(i * 16, 16)
            ixs[sl] = ixs[sl] + bias
            ixd[sl] = ixd[sl] + bias
        pltpu.async_copy(b1p_hbm.at[ixs], b1b, st_[7])
        pltpu.async_copy(b2p_hbm.at[ixd], b2b, st_[7])
        pltpu.async_copy(
            cp_hbm.at[pl.ds((gg * ROWS_PAD + base + k) * CHUNK, CHUNK)],
            cb0 if st_ is sets[0] else cb1, st_[7])

    def wait_gathers(k, gg, st_):
        pltpu.make_async_copy(b1p_hbm.at[st_[0]], st_[2], st_[7]).wait()
        pltpu.make_async_copy(b2p_hbm.at[st_[1]], st_[3], st_[7]).wait()
        pltpu.make_async_copy(
            cp_hbm.at[pl.ds((gg * ROWS_PAD + base + k) * CHUNK, CHUNK)],
            st_[4], st_[7]).wait()

    def compute(k, st_, acc):
        b1b, b2b, cb, xb = st_[2], st_[3], st_[4], st_[5]

        def row(i, carry):
            t0, u0 = carry
            sl0 = pl.ds(0, 16)
            x0 = b1b[i, sl0] + b2b[i, sl0] + cb[i, sl0]
            xb[i, sl0] = x0
            return (t0 + x0, u0 + x0 * x0)

        return lax.fori_loop(0, CHUNK, row, acc, unroll=8)

    def issue_write(k, gg, st_):
        pltpu.async_copy(
            st_[5],
            xp_hbm.at[pl.ds((gg * ROWS_PAD + base + k) * CHUNK, CHUNK)],
            st_[8])

    def wait_write(k, gg, st_):
        pltpu.make_async_copy(
            st_[5],
            xp_hbm.at[pl.ds((gg * ROWS_PAD + base + k) * CHUNK, CHUNK)],
            st_[8]).wait()

    for g in range(4):
        gg = c * 4 + g
        z = jnp.zeros((16,), jnp.float32)
        acc = (z, z)
        issue_idx(0, sets[0])
        bias_and_gather(0, gg, sets[0])
        issue_idx(1, sets[1])

        def pair(k2, acc):
            accv = acc
            for b in range(2):
                k = k2 * 2 + b
                st_ = sets[b]
                ot = sets[1 - b]
                wait_gathers(k, gg, st_)

                @pl.when(k >= 2)
                def _():
                    wait_write(k - 2, gg, st_)

                accv = compute(k, st_, accv)
                issue_write(k, gg, st_)
                bias_and_gather(k + 1, gg, ot)

                @pl.when(k + 2 < ROWS_PT)
                def _():
                    issue_idx(k + 2, st_)
            return accv

        acc = lax.fori_loop(0, (ROWS_PT - 2) // 2, pair, acc)
        for b in range(2):
            k = ROWS_PT - 2 + b
            st_ = sets[b]
            wait_gathers(k, gg, st_)
            wait_write(k - 2, gg, st_)
            acc = compute(k, st_, acc)
            issue_write(k, gg, st_)
            if b == 0:
                ot = sets[1]
                pltpu.make_async_copy(src2_hbm.at[base + k + 1], ot[0],
                                      ot[6]).wait()
                pltpu.make_async_copy(dst2_hbm.at[base + k + 1], ot[1],
                                      ot[6]).wait()
                bias = gg * N_TBL
                for i in range(CHUNK // 16):
                    sl = pl.ds(i * 16, 16)
                    ot[0][sl] = ot[0][sl] + bias
                    ot[1][sl] = ot[1][sl] + bias
                pltpu.async_copy(b1p_hbm.at[ot[0]], ot[2], ot[7])
                pltpu.async_copy(b2p_hbm.at[ot[1]], ot[3], ot[7])
                pltpu.async_copy(
                    cp_hbm.at[pl.ds((gg * ROWS_PAD + base + k + 1) * CHUNK,
                                    CHUNK)], ot[4], ot[7])
        for b in range(2):
            wait_write(ROWS_PT - 2 + b, gg, sets[b])

        stb[pl.ds(0, 16)] = acc[0]
        stb[pl.ds(16, 16)] = acc[1]
        pltpu.sync_copy(stb, st_hbm.at[pl.ds((gg * 16 + s) * 32, 32)])


def _stats_call(cp, b1p, b2p, src2, dst2):
    mesh = plsc.VectorSubcoreMesh(core_axis_name="c", subcore_axis_name="s")
    idx_t = pltpu.VMEM((CHUNK,), jnp.int32)
    buf_t = pltpu.VMEM((CHUNK, CG), jnp.float32)
    f = pl.kernel(
        _stats_body,
        out_type=(jax.ShapeDtypeStruct((NG * E_PAD, CG), jnp.float32),
                  jax.ShapeDtypeStruct((NG * 16 * 32,), jnp.float32)),
        mesh=mesh,
        scratch_types=[
            idx_t, idx_t, buf_t, buf_t, buf_t, buf_t,
            idx_t, idx_t, buf_t, buf_t, buf_t, buf_t,
            pltpu.VMEM((32,), jnp.float32),
            pltpu.SemaphoreType.DMA, pltpu.SemaphoreType.DMA,
            pltpu.SemaphoreType.DMA, pltpu.SemaphoreType.DMA,
            pltpu.SemaphoreType.DMA, pltpu.SemaphoreType.DMA,
        ],
        compiler_params=pltpu.CompilerParams(use_tc_tiling_on_sc=False),
    )
    return f(cp, b1p, b2p, src2, dst2)


def _agg_body(xp_hbm, ep_hbm, srt_hbm, a2p_hbm, a3p_hbm, src2_hbm, dst2_hbm,
              nf_hbm, df_hbm, nb_hbm, db_hbm,
              fwd_p, fwd_d, bwd_p, bwd_d,
              ixs0, ixd0, gxs0, gxd0, sxs0, sxd0, a2b0, a3b0, xb0, eb0, sgb0,
              ixs1, ixd1, gxs1, gxd1, sxs1, sxd1, a2b1, a3b1, xb1, eb1, sgb1,
              srtb, flb, zb,
              sem_i0, sem_i1, sem_g0, sem_g1, sem_s0, sem_s1):
    c = lax.axis_index("c")
    s = lax.axis_index("s")
    base = s * ROWS_PT
    row0 = s * STRIPE
    sets = ((ixs0, ixd0, gxs0, gxd0, sxs0, sxd0, a2b0, a3b0, xb0, eb0, sgb0,
             sem_i0, sem_g0, sem_s0),
            (ixs1, ixd1, gxs1, gxd1, sxs1, sxd1, a2b1, a3b1, xb1, eb1, sgb1,
             sem_i1, sem_g1, sem_s1))
    (IXS, IXD, GXS, GXD, SXS, SXD, A2B, A3B, XB, EB, SGB,
     SI, SG, SS) = range(14)

    def issue_idx(k, st_):
        pltpu.async_copy(src2_hbm.at[base + k], st_[IXS], st_[SI])
        pltpu.async_copy(dst2_hbm.at[base + k], st_[IXD], st_[SI])

    def bias_and_gather(k, gg, st_):
        pltpu.make_async_copy(src2_hbm.at[base + k], st_[IXS], st_[SI]).wait()
        pltpu.make_async_copy(dst2_hbm.at[base + k], st_[IXD], st_[SI]).wait()
        bias = gg * N_TBL
        for i in range(CHUNK // 16):
            sl = pl.ds(i * 16, 16)
            rs = st_[IXS][sl]
            rd = st_[IXD][sl]
            st_[GXS][sl] = rs + bias
            st_[GXD][sl] = rd + bias
            st_[SXS][sl] = rs
            st_[SXD][sl] = rd
        pltpu.async_copy(a2p_hbm.at[st_[GXS]], st_[A2B], st_[SG])
        pltpu.async_copy(a3p_hbm.at[st_[GXD]], st_[A3B], st_[SG])
        off = pl.ds((gg * ROWS_PAD + base + k) * CHUNK, CHUNK)
        pltpu.async_copy(xp_hbm.at[off], st_[XB], st_[SG])
        pltpu.async_copy(ep_hbm.at[off], st_[EB], st_[SG])

    def wait_gathers(k, gg, st_):
        pltpu.make_async_copy(a2p_hbm.at[st_[GXS]], st_[A2B], st_[SG]).wait()
        pltpu.make_async_copy(a3p_hbm.at[st_[GXD]], st_[A3B], st_[SG]).wait()
        off = pl.ds((gg * ROWS_PAD + base + k) * CHUNK, CHUNK)
        pltpu.make_async_copy(xp_hbm.at[off], st_[XB], st_[SG]).wait()
        pltpu.make_async_copy(ep_hbm.at[off], st_[EB], st_[SG]).wait()

    def compute(st_):
        a2b, a3b, xb, eb, sgb = (st_[A2B], st_[A3B], st_[XB], st_[EB],
                                 st_[SGB])

        def row(i, carry):
            sl = pl.ds(0, 16)
            sv = srtb[pl.ds(0, 16)]
            tv = srtb[pl.ds(16, 16)]
            z = jnp.maximum(xb[i, sl] * sv + tv, 0.0) + eb[i, sl]
            sg = 1.0 / (1.0 + jnp.exp(-z))
            sgb[i, sl] = sg
            a2b[i, sl] = a2b[i, sl] * sg
            a3b[i, sl] = a3b[i, sl] * sg
            return carry

        lax.fori_loop(0, CHUNK, row, 0, unroll=8)

    def issue_scatters(st_):
        pltpu.async_copy(st_[A2B], fwd_p.at[st_[SXD]], st_[SS], add=True)
        pltpu.async_copy(st_[SGB], fwd_d.at[st_[SXD]], st_[SS], add=True)
        pltpu.async_copy(st_[A3B], bwd_p.at[st_[SXS]], st_[SS], add=True)
        pltpu.async_copy(st_[SGB], bwd_d.at[st_[SXS]], st_[SS], add=True)

    def wait_scatters(st_):
        pltpu.make_async_copy(st_[A2B], fwd_p.at[st_[SXD]], st_[SS]).wait()
        pltpu.make_async_copy(st_[SGB], fwd_d.at[st_[SXD]], st_[SS]).wait()
        pltpu.make_async_copy(st_[A3B], bwd_p.at[st_[SXS]], st_[SS]).wait()
        pltpu.make_async_copy(st_[SGB], bwd_d.at[st_[SXS]], st_[SS]).wait()

    zrow = jnp.zeros((16,), jnp.float32)

    def zro(i, carry):
        zb[i, pl.ds(0, 16)] = zrow
        return carry

    lax.fori_loop(0, STRIPE, zro, 0, unroll=8)

    for g in range(4):
        gg = c * 4 + g
        pltpu.sync_copy(srt_hbm.at[pl.ds(gg * 32, 32)], srtb)

        for acc in (fwd_p, fwd_d, bwd_p, bwd_d):
            pltpu.sync_copy(zb, acc.at[pl.ds(row0, STRIPE)])
        plsc.subcore_barrier()

        issue_idx(0, sets[0])
        bias_and_gather(0, gg, sets[0])
        issue_idx(1, sets[1])

        def pair(k2, carry):
            for b in range(2):
                k = k2 * 2 + b
                st_ = sets[b]
                ot = sets[1 - b]
                wait_gathers(k, gg, st_)
                compute(st_)
                issue_scatters(st_)

                @pl.when(k >= 1)
                def _():
                    wait_scatters(ot)

                bias_and_gather(k + 1, gg, ot)

                @pl.when(k + 2 < ROWS_PT)
                def _():
                    issue_idx(k + 2, st_)
            return carry

        lax.fori_loop(0, (ROWS_PT - 2) // 2, pair, 0)
        for b in range(2):
            k = ROWS_PT - 2 + b
            st_ = sets[b]
            ot = sets[1 - b]
            wait_gathers(k, gg, st_)
            compute(st_)
            issue_scatters(st_)
            wait_scatters(ot)
            if b == 0:
                pltpu.make_async_copy(src2_hbm.at[base + k + 1], ot[IXS],
                                      ot[SI]).wait()
                pltpu.make_async_copy(dst2_hbm.at[base + k + 1], ot[IXD],
                                      ot[SI]).wait()
                bias = gg * N_TBL
                for i in range(CHUNK // 16):
                    sl = pl.ds(i * 16, 16)
                    rs = ot[IXS][sl]
                    rd = ot[IXD][sl]
                    ot[GXS][sl] = rs + bias
                    ot[GXD][sl] = rd + bias
                    ot[SXS][sl] = rs
                    ot[SXD][sl] = rd
                pltpu.async_copy(a2p_hbm.at[ot[GXS]], ot[A2B], ot[SG])
                pltpu.async_copy(a3p_hbm.at[ot[GXD]], ot[A3B], ot[SG])
                off = pl.ds((gg * ROWS_PAD + base + k + 1) * CHUNK, CHUNK)
                pltpu.async_copy(xp_hbm.at[off], ot[XB], ot[SG])
                pltpu.async_copy(ep_hbm.at[off], ot[EB], ot[SG])
        wait_scatters(sets[1])
        plsc.subcore_barrier()

        out_base = gg * N_PAD
        for acc, out in ((fwd_p, nf_hbm), (fwd_d, df_hbm),
                         (bwd_p, nb_hbm), (bwd_d, db_hbm)):
            pltpu.sync_copy(acc.at[pl.ds(row0, STRIPE)], flb)
            pltpu.sync_copy(flb, out.at[pl.ds(out_base + row0, STRIPE)])
        plsc.subcore_barrier()


def _agg_call(xp, ep, srt, a2p, a3p, src2, dst2):
    out4 = jax.ShapeDtypeStruct((NG * N_PAD, CG), jnp.float32)
    mesh = plsc.VectorSubcoreMesh(core_axis_name="c", subcore_axis_name="s")
    idx_t = pltpu.VMEM((CHUNK,), jnp.int32)
    buf_t = pltpu.VMEM((CHUNK, CG), jnp.float32)
    ring = [idx_t, idx_t, idx_t, idx_t, idx_t, idx_t,
            buf_t, buf_t, buf_t, buf_t, buf_t]
    f = pl.kernel(
        _agg_body,
        out_type=(out4, out4, out4, out4),
        mesh=mesh,
        scratch_types=[
            pltpu.VMEM_SHARED((N_PAD, CG), jnp.float32),
            pltpu.VMEM_SHARED((N_PAD, CG), jnp.float32),
            pltpu.VMEM_SHARED((N_PAD, CG), jnp.float32),
            pltpu.VMEM_SHARED((N_PAD, CG), jnp.float32),
        ] + ring + ring + [
            pltpu.VMEM((32,), jnp.float32),
            pltpu.VMEM((STRIPE, CG), jnp.float32),
            pltpu.VMEM((STRIPE, CG), jnp.float32),
            pltpu.SemaphoreType.DMA, pltpu.SemaphoreType.DMA,
            pltpu.SemaphoreType.DMA, pltpu.SemaphoreType.DMA,
            pltpu.SemaphoreType.DMA, pltpu.SemaphoreType.DMA,
        ],
        compiler_params=pltpu.CompilerParams(use_tc_tiling_on_sc=False),
    )
    return f(xp, ep, srt, a2p, a3p, src2, dst2)


def _pack_table(t):
    tp = jnp.concatenate([t, jnp.zeros((N_TBL - N_NODES, D), jnp.float32)], 0)
    return jnp.concatenate(
        [tp[:, i * CG:(i + 1) * CG] for i in range(NG)], 0)


def kernel(h, e, edge_index, WA1, bA1, WA2, bA2, WA3, bA3, WB1, bB1, WB2, bB2,
           WB3, bB3, gamma_h, beta_h, gamma_e, beta_e):
    src = edge_index[0].astype(jnp.int32)
    dst = edge_index[1].astype(jnp.int32)

    wstk = jnp.stack([WA1, WA2, WA3, WB1, WB2])
    bstk = jnp.stack([bA1, bA2, bA3, bB1, bB2])
    nmm = jax.ShapeDtypeStruct((N_NODES, D), jnp.float32)
    A1h, A2h, A3h, B1h, B2h = pl.pallas_call(
        _node_mm_kernel,
        out_shape=(nmm, nmm, nmm, nmm, nmm),
    )(h, wstk, bstk)

    cp, ep = pl.pallas_call(
        _edge_pack_kernel,
        grid=(NEB,),
        in_specs=[
            pl.BlockSpec((BE, D), lambda i: (jnp.minimum(i, N_EDGES // BE - 1),
                                             0)),
            pl.BlockSpec((D, D), lambda i: (0, 0)),
            pl.BlockSpec((1, D), lambda i: (0, 0)),
        ],
        out_specs=[
            pl.BlockSpec((NG, BE, CG), lambda i: (0, i, 0)),
            pl.BlockSpec((NG, BE, CG), lambda i: (0, i, 0)),
        ],
        out_shape=(jax.ShapeDtypeStruct((NG, E_PAD, CG), jnp.float32),
                   jax.ShapeDtypeStruct((NG, E_PAD, CG), jnp.float32)),
    )(e, WB3, bB3.reshape(1, D))
    cp = cp.reshape(NG * E_PAD, CG)
    ep = ep.reshape(NG * E_PAD, CG)

    pad = jnp.full((ROWS_PAD * CHUNK - N_EDGES,), N_NODES, jnp.int32)
    src2 = jnp.concatenate([src, pad]).reshape(ROWS_PAD, CHUNK)
    dst2 = jnp.concatenate([dst, pad]).reshape(ROWS_PAD, CHUNK)

    b1p = _pack_table(B1h)
    b2p = _pack_table(B2h)
    a2p = _pack_table(A2h)
    a3p = _pack_table(A3h)

    xp, st = _stats_call(cp, b1p, b2p, src2, dst2)

    stv = st.reshape(NG, 16, 2, CG)
    tot = stv.sum(axis=1)
    mu = tot[:, 0].reshape(D) / N_EDGES
    var = tot[:, 1].reshape(D) / N_EDGES - mu * mu
    sc = gamma_e / jnp.sqrt(var + 1e-5)
    tc_ = beta_e - mu * sc
    srt = jnp.concatenate(
        [jnp.stack([sc[i * CG:(i + 1) * CG], tc_[i * CG:(i + 1) * CG]])
         for i in range(NG)], 0).reshape(NG * 32)

    nf_p, df_p, nb_p, db_p = _agg_call(xp, ep, srt, a2p, a3p, src2, dst2)

    def unpack(t):
        return jnp.concatenate(
            [t[i * N_PAD:i * N_PAD + N_NODES] for i in range(NG)], axis=1)

    h_out = pl.pallas_call(
        _final_node_kernel,
        out_shape=jax.ShapeDtypeStruct(h.shape, h.dtype),
    )(A1h, unpack(nf_p), unpack(df_p), unpack(nb_p), unpack(db_p), h,
      gamma_h.reshape(1, -1), beta_h.reshape(1, -1))
    return (h_out, e)


# speedup vs baseline: 1.1376x; 1.1376x over previous
import functools

import jax
import jax.numpy as jnp
from jax import lax
from jax.experimental import pallas as pl
from jax.experimental.pallas import tpu as pltpu
from jax.experimental.pallas import tpu_sc as plsc

N_NODES = 10000
N_TBL = 10016
N_EDGES = 320000
D = 128

CG = 16
NG = D // CG
CHUNK = 128
ROWS = N_EDGES // CHUNK
ROWS_PT = 158
ROWS_PAD = 16 * ROWS_PT
E_PAD = ROWS_PAD * CHUNK
N_PAD = 10240
STRIPE = N_PAD // 16

BE = 512
NEB = E_PAD // BE


def _node_mm_kernel(h_ref, w_ref, b_ref, a1_ref, a2_ref, a3_ref, b1_ref,
                    b2_ref):
    h = h_ref[...]
    outs = (a1_ref, a2_ref, a3_ref, b1_ref, b2_ref)
    for i, o_ref in enumerate(outs):
        w = w_ref[i]
        o_ref[...] = jax.lax.dot_general(
            h, w, (((1,), (1,)), ((), ())),
            preferred_element_type=jnp.float32) + b_ref[i]


def _edge_pack_kernel(e_ref, w_ref, b_ref, cp_ref, ep_ref):
    i = pl.program_id(0)

    @pl.when(i < N_EDGES // BE)
    def _():
        ev = e_ref[...]
        c = jax.lax.dot_general(ev, w_ref[...], (((1,), (1,)), ((), ())),
                                preferred_element_type=jnp.float32) + b_ref[...]
        cp_ref[...] = c.reshape(BE, NG, CG).transpose(1, 0, 2)
        ep_ref[...] = ev.reshape(BE, NG, CG).transpose(1, 0, 2)

    @pl.when(i >= N_EDGES // BE)
    def _():
        cp_ref[...] = jnp.zeros((NG, BE, CG), jnp.float32)
        ep_ref[...] = jnp.zeros((NG, BE, CG), jnp.float32)


def _final_node_kernel(a1_ref, nf_ref, df_ref, nb_ref, db_ref, h_ref,
                       gamma_ref, beta_ref, out_ref):
    eps = 1e-6
    hf = nf_ref[...] / (df_ref[...] + eps)
    hb = nb_ref[...] / (db_ref[...] + eps)
    x = a1_ref[...] + hf + hb
    mu = jnp.mean(x, axis=0, keepdims=True)
    var = jnp.mean((x - mu) ** 2, axis=0, keepdims=True)
    xn = gamma_ref[...] * (x - mu) / jnp.sqrt(var + 1e-5) + beta_ref[...]
    out_ref[...] = jnp.maximum(xn, 0.0) + h_ref[...]


def _stats_body(cp_hbm, b1p_hbm, b2p_hbm, src2_hbm, dst2_hbm,
                xp_hbm, st_hbm,
                ixs0, ixd0, b1b0, b2b0, cb0, xb0,
                ixs1, ixd1, b1b1, b2b1, cb1, xb1,
                stb, sem_i0, sem_i1, sem_g0, sem_g1, sem_w0, sem_w1):
    c = lax.axis_index("c")
    s = lax.axis_index("s")
    base = s * ROWS_PT
    sets = ((ixs0, ixd0, b1b0, b2b0, cb0, xb0, sem_i0, sem_g0, sem_w0),
            (ixs1, ixd1, b1b1, b2b1, cb1, xb1, sem_i1, sem_g1, sem_w1))

    def issue_idx(k, st_):
        ixs, ixd = st_[0], st_[1]
        pltpu.async_copy(src2_hbm.at[base + k], ixs, st_[6])
        pltpu.async_copy(dst2_hbm.at[base + k], ixd, st_[6])

    def bias_and_gather(k, gg, st_):
        ixs, ixd, b1b, b2b = st_[0], st_[1], st_[2], st_[3]
        pltpu.make_async_copy(src2_hbm.at[base + k], ixs, st_[6]).wait()
        pltpu.make_async_copy(dst2_hbm.at[base + k], ixd, st_[6]).wait()
        bias = gg * N_TBL
        for i in range(CHUNK // 16):
            sl = pl.ds(i * 16, 16)
            ixs[sl] = ixs[sl] + bias
            ixd[sl] = ixd[sl] + bias
        pltpu.async_copy(b1p_hbm.at[ixs], b1b, st_[7])
        pltpu.async_copy(b2p_hbm.at[ixd], b2b, st_[7])
        pltpu.async_copy(
            cp_hbm.at[pl.ds((gg * ROWS_PAD + base + k) * CHUNK, CHUNK)],
            cb0 if st_ is sets[0] else cb1, st_[7])

    def wait_gathers(k, gg, st_):
        pltpu.make_async_copy(b1p_hbm.at[st_[0]], st_[2], st_[7]).wait()
        pltpu.make_async_copy(b2p_hbm.at[st_[1]], st_[3], st_[7]).wait()
        pltpu.make_async_copy(
            cp_hbm.at[pl.ds((gg * ROWS_PAD + base + k) * CHUNK, CHUNK)],
            st_[4], st_[7]).wait()

    def compute(k, st_, acc):
        b1b, b2b, cb, xb = st_[2], st_[3], st_[4], st_[5]

        def row(i, carry):
            t0, u0 = carry
            sl0 = pl.ds(0, 16)
            x0 = b1b[i, sl0] + b2b[i, sl0] + cb[i, sl0]
            xb[i, sl0] = x0
            return (t0 + x0, u0 + x0 * x0)

        return lax.fori_loop(0, CHUNK, row, acc, unroll=8)

    def issue_write(k, gg, st_):
        pltpu.async_copy(
            st_[5],
            xp_hbm.at[pl.ds((gg * ROWS_PAD + base + k) * CHUNK, CHUNK)],
            st_[8])

    def wait_write(k, gg, st_):
        pltpu.make_async_copy(
            st_[5],
            xp_hbm.at[pl.ds((gg * ROWS_PAD + base + k) * CHUNK, CHUNK)],
            st_[8]).wait()

    for g in range(4):
        gg = c * 4 + g
        z = jnp.zeros((16,), jnp.float32)
        acc = (z, z)
        issue_idx(0, sets[0])
        bias_and_gather(0, gg, sets[0])
        issue_idx(1, sets[1])

        def pair(k2, acc):
            accv = acc
            for b in range(2):
                k = k2 * 2 + b
                st_ = sets[b]
                ot = sets[1 - b]
                wait_gathers(k, gg, st_)

                @pl.when(k >= 2)
                def _():
                    wait_write(k - 2, gg, st_)

                bias_and_gather(k + 1, gg, ot)

                @pl.when(k + 2 < ROWS_PT)
                def _():
                    issue_idx(k + 2, st_)

                accv = compute(k, st_, accv)
                issue_write(k, gg, st_)
            return accv

        acc = lax.fori_loop(0, (ROWS_PT - 2) // 2, pair, acc)
        for b in range(2):
            k = ROWS_PT - 2 + b
            st_ = sets[b]
            wait_gathers(k, gg, st_)
            wait_write(k - 2, gg, st_)
            if b == 0:
                bias_and_gather(k + 1, gg, sets[1])
            acc = compute(k, st_, acc)
            issue_write(k, gg, st_)
        for b in range(2):
            wait_write(ROWS_PT - 2 + b, gg, sets[b])

        stb[pl.ds(0, 16)] = acc[0]
        stb[pl.ds(16, 16)] = acc[1]
        pltpu.sync_copy(stb, st_hbm.at[pl.ds((gg * 16 + s) * 32, 32)])


def _stats_call(cp, b1p, b2p, src2, dst2):
    mesh = plsc.VectorSubcoreMesh(core_axis_name="c", subcore_axis_name="s")
    idx_t = pltpu.VMEM((CHUNK,), jnp.int32)
    buf_t = pltpu.VMEM((CHUNK, CG), jnp.float32)
    f = pl.kernel(
        _stats_body,
        out_type=(jax.ShapeDtypeStruct((NG * E_PAD, CG), jnp.float32),
                  jax.ShapeDtypeStruct((NG * 16 * 32,), jnp.float32)),
        mesh=mesh,
        scratch_types=[
            idx_t, idx_t, buf_t, buf_t, buf_t, buf_t,
            idx_t, idx_t, buf_t, buf_t, buf_t, buf_t,
            pltpu.VMEM((32,), jnp.float32),
            pltpu.SemaphoreType.DMA, pltpu.SemaphoreType.DMA,
            pltpu.SemaphoreType.DMA, pltpu.SemaphoreType.DMA,
            pltpu.SemaphoreType.DMA, pltpu.SemaphoreType.DMA,
        ],
        compiler_params=pltpu.CompilerParams(use_tc_tiling_on_sc=False),
    )
    return f(cp, b1p, b2p, src2, dst2)


def _agg_body(xp_hbm, ep_hbm, srt_hbm, a2p_hbm, a3p_hbm, src2_hbm, dst2_hbm,
              nf_hbm, df_hbm, nb_hbm, db_hbm,
              fwd_p, fwd_d, bwd_p, bwd_d,
              ixs0, ixd0, gxs0, gxd0, sxs0, sxd0, a2b0, a3b0, xb0, eb0, sgb0,
              ixs1, ixd1, gxs1, gxd1, sxs1, sxd1, a2b1, a3b1, xb1, eb1, sgb1,
              srtb, flb, zb,
              sem_i0, sem_i1, sem_g0, sem_g1, sem_s0, sem_s1):
    c = lax.axis_index("c")
    s = lax.axis_index("s")
    base = s * ROWS_PT
    row0 = s * STRIPE
    sets = ((ixs0, ixd0, gxs0, gxd0, sxs0, sxd0, a2b0, a3b0, xb0, eb0, sgb0,
             sem_i0, sem_g0, sem_s0),
            (ixs1, ixd1, gxs1, gxd1, sxs1, sxd1, a2b1, a3b1, xb1, eb1, sgb1,
             sem_i1, sem_g1, sem_s1))
    (IXS, IXD, GXS, GXD, SXS, SXD, A2B, A3B, XB, EB, SGB,
     SI, SG, SS) = range(14)

    def issue_idx(k, st_):
        pltpu.async_copy(src2_hbm.at[base + k], st_[IXS], st_[SI])
        pltpu.async_copy(dst2_hbm.at[base + k], st_[IXD], st_[SI])

    def bias_and_gather(k, gg, st_):
        pltpu.make_async_copy(src2_hbm.at[base + k], st_[IXS], st_[SI]).wait()
        pltpu.make_async_copy(dst2_hbm.at[base + k], st_[IXD], st_[SI]).wait()
        bias = gg * N_TBL
        for i in range(CHUNK // 16):
            sl = pl.ds(i * 16, 16)
            rs = st_[IXS][sl]
            rd = st_[IXD][sl]
            st_[GXS][sl] = rs + bias
            st_[GXD][sl] = rd + bias
            st_[SXS][sl] = rs
            st_[SXD][sl] = rd
        pltpu.async_copy(a2p_hbm.at[st_[GXS]], st_[A2B], st_[SG])
        pltpu.async_copy(a3p_hbm.at[st_[GXD]], st_[A3B], st_[SG])
        off = pl.ds((gg * ROWS_PAD + base + k) * CHUNK, CHUNK)
        pltpu.async_copy(xp_hbm.at[off], st_[XB], st_[SG])
        pltpu.async_copy(ep_hbm.at[off], st_[EB], st_[SG])

    def wait_gathers(k, gg, st_):
        pltpu.make_async_copy(a2p_hbm.at[st_[GXS]], st_[A2B], st_[SG]).wait()
        pltpu.make_async_copy(a3p_hbm.at[st_[GXD]], st_[A3B], st_[SG]).wait()
        off = pl.ds((gg * ROWS_PAD + base + k) * CHUNK, CHUNK)
        pltpu.make_async_copy(xp_hbm.at[off], st_[XB], st_[SG]).wait()
        pltpu.make_async_copy(ep_hbm.at[off], st_[EB], st_[SG]).wait()

    def compute(st_):
        a2b, a3b, xb, eb, sgb = (st_[A2B], st_[A3B], st_[XB], st_[EB],
                                 st_[SGB])

        def row(i, carry):
            sl = pl.ds(0, 16)
            sv = srtb[pl.ds(0, 16)]
            tv = srtb[pl.ds(16, 16)]
            z = jnp.maximum(xb[i, sl] * sv + tv, 0.0) + eb[i, sl]
            sg = 1.0 / (1.0 + jnp.exp(-z))
            sgb[i, sl] = sg
            a2b[i, sl] = a2b[i, sl] * sg
            a3b[i, sl] = a3b[i, sl] * sg
            return carry

        lax.fori_loop(0, CHUNK, row, 0, unroll=8)

    def issue_scatters(st_):
        pltpu.async_copy(st_[A2B], fwd_p.at[st_[SXD]], st_[SS], add=True)
        pltpu.async_copy(st_[SGB], fwd_d.at[st_[SXD]], st_[SS], add=True)
        pltpu.async_copy(st_[A3B], bwd_p.at[st_[SXS]], st_[SS], add=True)
        pltpu.async_copy(st_[SGB], bwd_d.at[st_[SXS]], st_[SS], add=True)

    def wait_scatters(st_):
        pltpu.make_async_copy(st_[A2B], fwd_p.at[st_[SXD]], st_[SS]).wait()
        pltpu.make_async_copy(st_[SGB], fwd_d.at[st_[SXD]], st_[SS]).wait()
        pltpu.make_async_copy(st_[A3B], bwd_p.at[st_[SXS]], st_[SS]).wait()
        pltpu.make_async_copy(st_[SGB], bwd_d.at[st_[SXS]], st_[SS]).wait()

    zrow = jnp.zeros((16,), jnp.float32)

    def zro(i, carry):
        zb[i, pl.ds(0, 16)] = zrow
        return carry

    lax.fori_loop(0, STRIPE, zro, 0, unroll=8)

    for g in range(4):
        gg = c * 4 + g
        pltpu.sync_copy(srt_hbm.at[pl.ds(gg * 32, 32)], srtb)

        for acc in (fwd_p, fwd_d, bwd_p, bwd_d):
            pltpu.sync_copy(zb, acc.at[pl.ds(row0, STRIPE)])
        plsc.subcore_barrier()

        issue_idx(0, sets[0])
        bias_and_gather(0, gg, sets[0])
        issue_idx(1, sets[1])

        def pair(k2, carry):
            for b in range(2):
                k = k2 * 2 + b
                st_ = sets[b]
                ot = sets[1 - b]
                wait_gathers(k, gg, st_)

                @pl.when(k >= 1)
                def _():
                    wait_scatters(ot)

                bias_and_gather(k + 1, gg, ot)

                @pl.when(k + 2 < ROWS_PT)
                def _():
                    issue_idx(k + 2, st_)

                compute(st_)
                issue_scatters(st_)
            return carry

        lax.fori_loop(0, (ROWS_PT - 2) // 2, pair, 0)
        for b in range(2):
            k = ROWS_PT - 2 + b
            st_ = sets[b]
            ot = sets[1 - b]
            wait_gathers(k, gg, st_)
            wait_scatters(ot)
            if b == 0:
                bias_and_gather(k + 1, gg, sets[1])
            compute(st_)
            issue_scatters(st_)
        wait_scatters(sets[1])
        plsc.subcore_barrier()

        out_base = gg * N_PAD
        for acc, out in ((fwd_p, nf_hbm), (fwd_d, df_hbm),
                         (bwd_p, nb_hbm), (bwd_d, db_hbm)):
            pltpu.sync_copy(acc.at[pl.ds(row0, STRIPE)], flb)
            pltpu.sync_copy(flb, out.at[pl.ds(out_base + row0, STRIPE)])
        plsc.subcore_barrier()


def _agg_call(xp, ep, srt, a2p, a3p, src2, dst2):
    out4 = jax.ShapeDtypeStruct((NG * N_PAD, CG), jnp.float32)
    mesh = plsc.VectorSubcoreMesh(core_axis_name="c", subcore_axis_name="s")
    idx_t = pltpu.VMEM((CHUNK,), jnp.int32)
    buf_t = pltpu.VMEM((CHUNK, CG), jnp.float32)
    ring = [idx_t, idx_t, idx_t, idx_t, idx_t, idx_t,
            buf_t, buf_t, buf_t, buf_t, buf_t]
    f = pl.kernel(
        _agg_body,
        out_type=(out4, out4, out4, out4),
        mesh=mesh,
        scratch_types=[
            pltpu.VMEM_SHARED((N_PAD, CG), jnp.float32),
            pltpu.VMEM_SHARED((N_PAD, CG), jnp.float32),
            pltpu.VMEM_SHARED((N_PAD, CG), jnp.float32),
            pltpu.VMEM_SHARED((N_PAD, CG), jnp.float32),
        ] + ring + ring + [
            pltpu.VMEM((32,), jnp.float32),
            pltpu.VMEM((STRIPE, CG), jnp.float32),
            pltpu.VMEM((STRIPE, CG), jnp.float32),
            pltpu.SemaphoreType.DMA, pltpu.SemaphoreType.DMA,
            pltpu.SemaphoreType.DMA, pltpu.SemaphoreType.DMA,
            pltpu.SemaphoreType.DMA, pltpu.SemaphoreType.DMA,
        ],
        compiler_params=pltpu.CompilerParams(use_tc_tiling_on_sc=False),
    )
    return f(xp, ep, srt, a2p, a3p, src2, dst2)


def _pack_table(t):
    tp = jnp.concatenate([t, jnp.zeros((N_TBL - N_NODES, D), jnp.float32)], 0)
    return jnp.concatenate(
        [tp[:, i * CG:(i + 1) * CG] for i in range(NG)], 0)


def kernel(h, e, edge_index, WA1, bA1, WA2, bA2, WA3, bA3, WB1, bB1, WB2, bB2,
           WB3, bB3, gamma_h, beta_h, gamma_e, beta_e):
    src = edge_index[0].astype(jnp.int32)
    dst = edge_index[1].astype(jnp.int32)

    wstk = jnp.stack([WA1, WA2, WA3, WB1, WB2])
    bstk = jnp.stack([bA1, bA2, bA3, bB1, bB2])
    nmm = jax.ShapeDtypeStruct((N_NODES, D), jnp.float32)
    A1h, A2h, A3h, B1h, B2h = pl.pallas_call(
        _node_mm_kernel,
        out_shape=(nmm, nmm, nmm, nmm, nmm),
    )(h, wstk, bstk)

    cp, ep = pl.pallas_call(
        _edge_pack_kernel,
        grid=(NEB,),
        in_specs=[
            pl.BlockSpec((BE, D), lambda i: (jnp.minimum(i, N_EDGES // BE - 1),
                                             0)),
            pl.BlockSpec((D, D), lambda i: (0, 0)),
            pl.BlockSpec((1, D), lambda i: (0, 0)),
        ],
        out_specs=[
            pl.BlockSpec((NG, BE, CG), lambda i: (0, i, 0)),
            pl.BlockSpec((NG, BE, CG), lambda i: (0, i, 0)),
        ],
        out_shape=(jax.ShapeDtypeStruct((NG, E_PAD, CG), jnp.float32),
                   jax.ShapeDtypeStruct((NG, E_PAD, CG), jnp.float32)),
    )(e, WB3, bB3.reshape(1, D))
    cp = cp.reshape(NG * E_PAD, CG)
    ep = ep.reshape(NG * E_PAD, CG)

    pad = jnp.full((ROWS_PAD * CHUNK - N_EDGES,), N_NODES, jnp.int32)
    src2 = jnp.concatenate([src, pad]).reshape(ROWS_PAD, CHUNK)
    dst2 = jnp.concatenate([dst, pad]).reshape(ROWS_PAD, CHUNK)

    b1p = _pack_table(B1h)
    b2p = _pack_table(B2h)
    a2p = _pack_table(A2h)
    a3p = _pack_table(A3h)

    xp, st = _stats_call(cp, b1p, b2p, src2, dst2)

    stv = st.reshape(NG, 16, 2, CG)
    tot = stv.sum(axis=1)
    mu = tot[:, 0].reshape(D) / N_EDGES
    var = tot[:, 1].reshape(D) / N_EDGES - mu * mu
    sc = gamma_e / jnp.sqrt(var + 1e-5)
    tc_ = beta_e - mu * sc
    srt = jnp.concatenate(
        [jnp.stack([sc[i * CG:(i + 1) * CG], tc_[i * CG:(i + 1) * CG]])
         for i in range(NG)], 0).reshape(NG * 32)

    nf_p, df_p, nb_p, db_p = _agg_call(xp, ep, srt, a2p, a3p, src2, dst2)

    def unpack(t):
        return jnp.concatenate(
            [t[i * N_PAD:i * N_PAD + N_NODES] for i in range(NG)], axis=1)

    h_out = pl.pallas_call(
        _final_node_kernel,
        out_shape=jax.ShapeDtypeStruct(h.shape, h.dtype),
    )(A1h, unpack(nf_p), unpack(df_p), unpack(nb_p), unpack(db_p), h,
      gamma_h.reshape(1, -1), beta_h.reshape(1, -1))
    return (h_out, e)
